# baseline (speedup 1.0000x reference)
import jax
import jax.numpy as jnp
from jax.experimental import pallas as pl

_B, _N, _M, _D, _K = 1024, 26, 26, 128, 10
_BB = 32
_L = _BB * _N


def _dot(a, b):
    return jax.lax.dot(a, b, preferred_element_type=jnp.float32)


def _attn_kernel(f_ref, c_ref, wq_ref, wk_ref, out_ref):
    wqT = wq_ref[...].T
    wkT = wk_ref[...].T
    f = jnp.concatenate([f_ref[b] for b in range(_BB)], axis=0)
    c = jnp.concatenate([c_ref[b] for b in range(_BB)], axis=0)

    q = _dot(f, wqT)
    k = _dot(c, wkT)
    W = jax.lax.dot_general(
        q, k, (((1,), (1,)), ((), ())),
        preferred_element_type=jnp.float32)

    S = jnp.concatenate(
        [W[_N * b:_N * (b + 1), _M * b:_M * (b + 1)] for b in range(_BB)],
        axis=0)

    vals = []
    wcur = S
    for _ in range(_K):
        mx = jnp.max(wcur, axis=1, keepdims=True)
        vals.append(mx)
        wcur = jnp.where(wcur == mx, -jnp.inf, wcur)

    exps = [jnp.exp(v - vals[0]) for v in vals]
    inv = 1.0 / sum(exps)

    row_iota = jax.lax.broadcasted_iota(jnp.int32, (_L, _L), 0)
    l_iota = jax.lax.broadcasted_iota(jnp.int32, (_L, _L), 1)
    Wm = jnp.where(row_iota // _N == l_iota // _M, W, -jnp.inf)
    for kk in range(_K):
        Pk = jnp.where(Wm == vals[kk], exps[kk] * inv, 0.0)
        Gk = _dot(Pk, c)
        Ok = Gk * f
        for b in range(_BB):
            out_ref[pl.ds(b, 1), pl.Slice(kk, _N, _K), :] = (
                Ok[_N * b:_N * (b + 1), :].reshape(1, _N, _D))


def kernel(featureVec, contextVec, Wq, Wk):
    return pl.pallas_call(
        _attn_kernel,
        grid=(_B // _BB,),
        in_specs=[
            pl.BlockSpec((_BB, _N, _D), lambda i: (i, 0, 0)),
            pl.BlockSpec((_BB, _M, _D), lambda i: (i, 0, 0)),
            pl.BlockSpec((_D, _D), lambda i: (0, 0)),
            pl.BlockSpec((_D, _D), lambda i: (0, 0)),
        ],
        out_specs=pl.BlockSpec((_BB, _N * _K, _D), lambda i: (i, 0, 0)),
        out_shape=jax.ShapeDtypeStruct((_B, _N * _K, _D), jnp.float32),
    )(featureVec, contextVec, Wq, Wk)


# speedup vs baseline: 1.0644x; 1.0644x over previous
import jax
import jax.numpy as jnp
from jax.experimental import pallas as pl

_B, _N, _M, _D, _K = 1024, 26, 26, 128, 10
_BB = 16
_L = _BB * _N


def _dot(a, b):
    return jax.lax.dot(a, b, preferred_element_type=jnp.float32)


def _attn_kernel(f_ref, c_ref, wq_ref, wk_ref, out_ref):
    wqT = wq_ref[...].T
    wkT = wk_ref[...].T
    f = jnp.concatenate([f_ref[b] for b in range(_BB)], axis=0)
    c = jnp.concatenate([c_ref[b] for b in range(_BB)], axis=0)

    q = _dot(f, wqT)
    k = _dot(c, wkT)
    W = jax.lax.dot_general(
        q, k, (((1,), (1,)), ((), ())),
        preferred_element_type=jnp.float32)

    S = jnp.concatenate(
        [W[_N * b:_N * (b + 1), _M * b:_M * (b + 1)] for b in range(_BB)],
        axis=0)

    vals = []
    wcur = S
    for _ in range(_K):
        mx = jnp.max(wcur, axis=1, keepdims=True)
        vals.append(mx)
        wcur = jnp.where(wcur == mx, -jnp.inf, wcur)

    exps = [jnp.exp(v - vals[0]) for v in vals]
    inv = 1.0 / sum(exps)

    row_iota = jax.lax.broadcasted_iota(jnp.int32, (_L, _L), 0)
    l_iota = jax.lax.broadcasted_iota(jnp.int32, (_L, _L), 1)
    Wm = jnp.where(row_iota // _N == l_iota // _M, W, -jnp.inf)
    for kk in range(_K):
        Pk = jnp.where(Wm == vals[kk], exps[kk] * inv, 0.0)
        Gk = _dot(Pk, c)
        Ok = Gk * f
        for b in range(_BB):
            out_ref[pl.ds(b, 1), pl.Slice(kk, _N, _K), :] = (
                Ok[_N * b:_N * (b + 1), :].reshape(1, _N, _D))


def kernel(featureVec, contextVec, Wq, Wk):
    return pl.pallas_call(
        _attn_kernel,
        grid=(_B // _BB,),
        in_specs=[
            pl.BlockSpec((_BB, _N, _D), lambda i: (i, 0, 0)),
            pl.BlockSpec((_BB, _M, _D), lambda i: (i, 0, 0)),
            pl.BlockSpec((_D, _D), lambda i: (0, 0)),
            pl.BlockSpec((_D, _D), lambda i: (0, 0)),
        ],
        out_specs=pl.BlockSpec((_BB, _N * _K, _D), lambda i: (i, 0, 0)),
        out_shape=jax.ShapeDtypeStruct((_B, _N * _K, _D), jnp.float32),
    )(featureVec, contextVec, Wq, Wk)
